# baseline (speedup 1.0000x reference)
import functools

import numpy as np
import jax
import jax.numpy as jnp
from jax import lax
from jax.experimental import pallas as pl
from jax.experimental.pallas import tpu as pltpu
from jax.experimental.pallas import tpu_sc as plsc

_NC = 2
_NS = 16
_NW = _NC * _NS
_LANES = 16
_NBUF = 4
_NOB = 2
_CH = 100


def _pos_encoding(dk: int, length: int) -> np.ndarray:
    i = np.arange(dk)
    even = (i % 2 == 0).astype(np.float64)
    power = 10000.0 ** (2.0 * i / dk)
    pos = np.arange(length)[:, None]
    ang = pos / power[None, :]
    return (np.sin(ang) * even + np.cos(ang) * (1.0 - even)).astype(np.float32)


@functools.partial(jax.jit, static_argnames=("interpret",))
def kernel(tokens, table, *, interpret=False):
    B, L = tokens.shape
    V, D = table.shape
    assert B % _NW == 0 and D % _LANES == 0 and V % 2 == 0
    spw = B // _NW
    ch = _CH
    assert L % ch == 0
    nch = L // ch
    nk = spw * nch
    assert nk % _NBUF == 0 and nk >= 2 * _NBUF
    gfull, gtail = ch // _LANES, ch % _LANES
    chp = (gfull + (1 if gtail else 0)) * _LANES

    pos = jnp.asarray(_pos_encoding(D, L))
    tok = tokens.astype(jnp.int32)
    idx2 = (tok >> 1).reshape(_NW, nk, ch)
    par = jnp.pad(
        (tok & 1).astype(jnp.float32).reshape(_NW, nk, ch),
        ((0, 0), (0, 0), (0, chp - ch))
    )
    t_even = table[0::2]
    t_odd = table[1::2]

    mesh = plsc.VectorSubcoreMesh(
        core_axis_name="c", subcore_axis_name="s",
        num_cores=_NC, num_subcores=_NS,
    )

    @functools.partial(
        pl.kernel,
        out_type=jax.ShapeDtypeStruct((B, L, D), jnp.float32),
        mesh=mesh,
        scratch_types=[
            pltpu.VMEM((nk, ch), jnp.int32),
            pltpu.VMEM((nk, chp), jnp.float32),
            pltpu.VMEM((_NBUF, ch, D), jnp.float32),
            pltpu.VMEM((_NBUF, ch, D), jnp.float32),
            pltpu.VMEM((_NOB, ch, D), jnp.float32),
            pltpu.VMEM((L, D), jnp.float32),
            [pltpu.SemaphoreType.DMA for _ in range(_NBUF)],
            [pltpu.SemaphoreType.DMA for _ in range(_NOB)],
        ],
        compiler_params=pltpu.CompilerParams(use_tc_tiling_on_sc=False),
        interpret=interpret,
    )
    def emb_kernel(idx_hbm, par_hbm, pos_hbm, te_hbm, to_hbm, out_hbm,
                   idx_all, par_all, rows_e, rows_d, rows_o, pos_v,
                   sem_g, sem_o):
        wid = lax.axis_index("s") * _NC + lax.axis_index("c")
        base = wid * spw
        pltpu.sync_copy(pos_hbm, pos_v)
        pltpu.sync_copy(idx_hbm.at[wid], idx_all)
        pltpu.sync_copy(par_hbm.at[wid], par_all)

        def gather(k, b, issue):
            mk = pltpu.async_copy if issue else pltpu.make_async_copy
            return [mk(te_hbm.at[idx_all.at[k]], rows_e.at[b], sem_g[b]),
                    mk(to_hbm.at[idx_all.at[k]], rows_d.at[b], sem_g[b])]

        def store(k, o, issue):
            mk = pltpu.async_copy if issue else pltpu.make_async_copy
            j = k // nch
            h = k % nch
            return mk(rows_o.at[o],
                      out_hbm.at[base + j, pl.ds(h * ch, ch)],
                      sem_o[o])

        for p in range(_NBUF - 1):
            gather(p, p, True)

        @pl.loop(0, nk, step=_NBUF)
        def _outer(ko):
            for b in range(_NBUF):
                k = ko + b
                for d in gather(k, b, False):
                    d.wait()
                o = b % _NOB

                @pl.when(k >= _NOB)
                def _():
                    store(k - _NOB, o, False).wait()

                h = k % nch
                r0 = h * ch

                def do_rows(g, n):
                    par16 = par_all[k, pl.ds(g * _LANES, _LANES)]
                    for i in range(n):
                        rr = g * _LANES + i
                        lane = jnp.full((_LANES,), i, jnp.int32)
                        pf = par16.at[lane].get(mode="promise_in_bounds")
                        for c in range(D // _LANES):
                            sl = pl.ds(c * _LANES, _LANES)
                            lo = rows_e[b, rr, sl]
                            hi = rows_d[b, rr, sl]
                            rows_o[o, rr, sl] = (
                                lo + (hi - lo) * pf + pos_v[r0 + rr, sl]
                            )

                @pl.loop(0, gfull)
                def _grp(g):
                    do_rows(g, _LANES)

                if gtail:
                    do_rows(gfull, gtail)

                bp = (b - 1) % _NBUF

                @pl.when(k + _NBUF - 1 < nk)
                def _():
                    gather(k + _NBUF - 1, bp, True)

                store(k, o, True)

        for t in range(_NOB):
            k = nk - _NOB + t
            store(k, (k % _NBUF) % _NOB, False).wait()

    return emb_kernel(idx2, par, pos, t_even, t_odd)


# speedup vs baseline: 2.9586x; 2.9586x over previous
import functools

import numpy as np
import jax
import jax.numpy as jnp
from jax import lax
from jax.experimental import pallas as pl
from jax.experimental.pallas import tpu as pltpu
from jax.experimental.pallas import tpu_sc as plsc

_NC = 2
_NS = 16
_NW = _NC * _NS
_LANES = 16
_NBUF = 4
_NOB = 2
_CH = 100


def _pos_encoding(dk: int, length: int) -> np.ndarray:
    i = np.arange(dk)
    even = (i % 2 == 0).astype(np.float64)
    power = 10000.0 ** (2.0 * i / dk)
    pos = np.arange(length)[:, None]
    ang = pos / power[None, :]
    return (np.sin(ang) * even + np.cos(ang) * (1.0 - even)).astype(np.float32)


@functools.partial(jax.jit, static_argnames=("interpret",))
def kernel(tokens, table, *, interpret=False):
    B, L = tokens.shape
    V, D = table.shape
    assert B % _NW == 0 and D % _LANES == 0 and V % 2 == 0
    spw = B // _NW
    ch = _CH
    assert L % ch == 0
    nch = L // ch
    nk = spw * nch
    assert nk % _NBUF == 0 and nk >= 2 * _NBUF
    gfull, gtail = ch // _LANES, ch % _LANES
    chp = (gfull + (1 if gtail else 0)) * _LANES

    pos = jnp.asarray(_pos_encoding(D, L))
    H = V // 2
    tok = tokens.astype(jnp.int32)
    idx_lo = jnp.minimum(tok, H - 1).reshape(_NW, nk, ch)
    idx_hi = jnp.maximum(tok - H, 0).reshape(_NW, nk, ch)
    par = jnp.pad(
        (tok >= H).astype(jnp.float32).reshape(_NW, nk, ch),
        ((0, 0), (0, 0), (0, chp - ch))
    )
    t_lo = table[:H]
    t_hi = table[H:]

    mesh = plsc.VectorSubcoreMesh(
        core_axis_name="c", subcore_axis_name="s",
        num_cores=_NC, num_subcores=_NS,
    )

    @functools.partial(
        pl.kernel,
        out_type=jax.ShapeDtypeStruct((B, L, D), jnp.float32),
        mesh=mesh,
        scratch_types=[
            pltpu.VMEM((nk, ch), jnp.int32),
            pltpu.VMEM((nk, ch), jnp.int32),
            pltpu.VMEM((nk, chp), jnp.float32),
            pltpu.VMEM((_NBUF, ch, D), jnp.float32),
            pltpu.VMEM((_NBUF, ch, D), jnp.float32),
            pltpu.VMEM((_NOB, ch, D), jnp.float32),
            pltpu.VMEM((L, D), jnp.float32),
            [pltpu.SemaphoreType.DMA for _ in range(_NBUF)],
            [pltpu.SemaphoreType.DMA for _ in range(_NOB)],
        ],
        compiler_params=pltpu.CompilerParams(use_tc_tiling_on_sc=False),
        interpret=interpret,
    )
    def emb_kernel(il_hbm, ih_hbm, par_hbm, pos_hbm, te_hbm, to_hbm,
                   out_hbm, il_all, ih_all, par_all, rows_e, rows_d,
                   rows_o, pos_v, sem_g, sem_o):
        wid = lax.axis_index("s") * _NC + lax.axis_index("c")
        base = wid * spw
        pltpu.sync_copy(pos_hbm, pos_v)
        pltpu.sync_copy(il_hbm.at[wid], il_all)
        pltpu.sync_copy(ih_hbm.at[wid], ih_all)
        pltpu.sync_copy(par_hbm.at[wid], par_all)

        def gather(k, b, issue):
            mk = pltpu.async_copy if issue else pltpu.make_async_copy
            return [mk(te_hbm.at[il_all.at[k]], rows_e.at[b], sem_g[b]),
                    mk(to_hbm.at[ih_all.at[k]], rows_d.at[b], sem_g[b])]

        def store(k, o, issue):
            mk = pltpu.async_copy if issue else pltpu.make_async_copy
            j = k // nch
            h = k % nch
            return mk(rows_o.at[o],
                      out_hbm.at[base + j, pl.ds(h * ch, ch)],
                      sem_o[o])

        for p in range(_NBUF - 1):
            gather(p, p, True)

        @pl.loop(0, nk, step=_NBUF)
        def _outer(ko):
            for b in range(_NBUF):
                k = ko + b
                for d in gather(k, b, False):
                    d.wait()
                o = b % _NOB

                @pl.when(k >= _NOB)
                def _():
                    store(k - _NOB, o, False).wait()

                h = k % nch
                r0 = h * ch

                def do_rows(g, n):
                    par16 = par_all[k, pl.ds(g * _LANES, _LANES)]
                    for i in range(n):
                        rr = g * _LANES + i
                        lane = jnp.full((_LANES,), i, jnp.int32)
                        pf = par16.at[lane].get(mode="promise_in_bounds")
                        for c in range(D // _LANES):
                            sl = pl.ds(c * _LANES, _LANES)
                            lo = rows_e[b, rr, sl]
                            hi = rows_d[b, rr, sl]
                            rows_o[o, rr, sl] = (
                                lo + (hi - lo) * pf + pos_v[r0 + rr, sl]
                            )

                @pl.loop(0, gfull)
                def _grp(g):
                    do_rows(g, _LANES)

                if gtail:
                    do_rows(gfull, gtail)

                bp = (b - 1) % _NBUF

                @pl.when(k + _NBUF - 1 < nk)
                def _():
                    gather(k + _NBUF - 1, bp, True)

                store(k, o, True)

        for t in range(_NOB):
            k = nk - _NOB + t
            store(k, (k % _NBUF) % _NOB, False).wait()

    return emb_kernel(idx_lo, idx_hi, par, pos, t_lo, t_hi)


# speedup vs baseline: 11.6115x; 3.9246x over previous
import functools

import numpy as np
import jax
import jax.numpy as jnp
from jax import lax
from jax.experimental import pallas as pl
from jax.experimental.pallas import tpu as pltpu
from jax.experimental.pallas import tpu_sc as plsc

_NC = 2
_NS = 16
_NW = _NC * _NS
_LANES = 16
_NBUF = 4


def _pos_encoding(dk: int, length: int) -> np.ndarray:
    i = np.arange(dk)
    even = (i % 2 == 0).astype(np.float64)
    power = 10000.0 ** (2.0 * i / dk)
    pos = np.arange(length)[:, None]
    ang = pos / power[None, :]
    return (np.sin(ang) * even + np.cos(ang) * (1.0 - even)).astype(np.float32)


@functools.partial(jax.jit, static_argnames=("interpret",))
def kernel(tokens, table, *, interpret=False):
    B, L = tokens.shape
    V, D = table.shape
    assert B % _NW == 0 and D % _LANES == 0
    spw = B // _NW
    assert spw % _NBUF == 0
    nch = -(-L // 128)
    assert L % nch == 0
    ch = L // nch

    pos = jnp.asarray(_pos_encoding(D, L))
    tokens_c = tokens.astype(jnp.int32).reshape(_NW, spw, nch, ch)

    mesh = plsc.VectorSubcoreMesh(
        core_axis_name="c", subcore_axis_name="s",
        num_cores=_NC, num_subcores=_NS,
    )

    @functools.partial(
        pl.kernel,
        out_type=jax.ShapeDtypeStruct((B, L, D), jnp.float32),
        mesh=mesh,
        scratch_types=[
            pltpu.VMEM((spw, nch, ch), jnp.int32),
            pltpu.VMEM((_NBUF, L, D), jnp.float32),
            pltpu.VMEM((L, D), jnp.float32),
            [pltpu.SemaphoreType.DMA for _ in range(_NBUF)],
        ],
        compiler_params=pltpu.CompilerParams(use_tc_tiling_on_sc=False),
        interpret=interpret,
    )
    def emb_kernel(tokens_hbm, pos_hbm, table_hbm, out_hbm,
                   idx_all, rows, pos_v, sem_g):
        wid = lax.axis_index("s") * _NC + lax.axis_index("c")
        base = wid * spw
        pltpu.sync_copy(pos_hbm, pos_v)
        pltpu.sync_copy(tokens_hbm.at[wid], idx_all)

        def gather(j, b, issue):
            mk = pltpu.async_copy if issue else pltpu.make_async_copy
            return [
                mk(table_hbm.at[idx_all.at[j, h]],
                   rows.at[b, pl.ds(h * ch, ch)], sem_g[b])
                for h in range(nch)
            ]

        for k in range(_NBUF - 1):
            gather(k, k, True)

        @pl.loop(0, spw, step=_NBUF)
        def _outer(jo):
            for b in range(_NBUF):
                j = jo + b
                for d in gather(j, b, False):
                    d.wait()

                @pl.loop(0, L)
                def _row(r):
                    for c in range(D // _LANES):
                        sl = pl.ds(c * _LANES, _LANES)
                        rows[b, r, sl] = rows[b, r, sl] + pos_v[r, sl]

                bp = (b - 1) % _NBUF

                @pl.when(j + _NBUF - 1 < spw)
                def _():
                    gather(j + _NBUF - 1, bp, True)

                pltpu.sync_copy(rows.at[b], out_hbm.at[base + j])

    return emb_kernel(tokens_c, pos, table)


# speedup vs baseline: 11.6132x; 1.0001x over previous
import functools

import numpy as np
import jax
import jax.numpy as jnp
from jax import lax
from jax.experimental import pallas as pl
from jax.experimental.pallas import tpu as pltpu
from jax.experimental.pallas import tpu_sc as plsc

_NC = 2
_NS = 16
_NW = _NC * _NS
_LANES = 16
_NBUF = 8


def _pos_encoding(dk: int, length: int) -> np.ndarray:
    i = np.arange(dk)
    even = (i % 2 == 0).astype(np.float64)
    power = 10000.0 ** (2.0 * i / dk)
    pos = np.arange(length)[:, None]
    ang = pos / power[None, :]
    return (np.sin(ang) * even + np.cos(ang) * (1.0 - even)).astype(np.float32)


@functools.partial(jax.jit, static_argnames=("interpret",))
def kernel(tokens, table, *, interpret=False):
    B, L = tokens.shape
    V, D = table.shape
    assert B % _NW == 0 and D % _LANES == 0
    spw = B // _NW
    assert spw % _NBUF == 0
    nch = -(-L // 128)
    assert L % nch == 0
    ch = L // nch

    pos = jnp.asarray(_pos_encoding(D, L))
    tokens_c = tokens.astype(jnp.int32).reshape(_NW, spw, nch, ch)

    mesh = plsc.VectorSubcoreMesh(
        core_axis_name="c", subcore_axis_name="s",
        num_cores=_NC, num_subcores=_NS,
    )

    @functools.partial(
        pl.kernel,
        out_type=jax.ShapeDtypeStruct((B, L, D), jnp.float32),
        mesh=mesh,
        scratch_types=[
            pltpu.VMEM((spw, nch, ch), jnp.int32),
            pltpu.VMEM((_NBUF, L, D), jnp.float32),
            pltpu.VMEM((L, D), jnp.float32),
            [pltpu.SemaphoreType.DMA for _ in range(_NBUF)],
        ],
        compiler_params=pltpu.CompilerParams(use_tc_tiling_on_sc=False),
        interpret=interpret,
    )
    def emb_kernel(tokens_hbm, pos_hbm, table_hbm, out_hbm,
                   idx_all, rows, pos_v, sem_g):
        wid = lax.axis_index("s") * _NC + lax.axis_index("c")
        base = wid * spw
        pltpu.sync_copy(pos_hbm, pos_v)
        pltpu.sync_copy(tokens_hbm.at[wid], idx_all)

        def gather(j, b, issue):
            mk = pltpu.async_copy if issue else pltpu.make_async_copy
            return [
                mk(table_hbm.at[idx_all.at[j, h]],
                   rows.at[b, pl.ds(h * ch, ch)], sem_g[b])
                for h in range(nch)
            ]

        for k in range(_NBUF - 1):
            gather(k, k, True)

        @pl.loop(0, spw, step=_NBUF)
        def _outer(jo):
            for b in range(_NBUF):
                j = jo + b
                for d in gather(j, b, False):
                    d.wait()

                @pl.loop(0, L)
                def _row(r):
                    for c in range(D // _LANES):
                        sl = pl.ds(c * _LANES, _LANES)
                        rows[b, r, sl] = rows[b, r, sl] + pos_v[r, sl]

                bp = (b - 1) % _NBUF

                @pl.when(j + _NBUF - 1 < spw)
                def _():
                    gather(j + _NBUF - 1, bp, True)

                pltpu.sync_copy(rows.at[b], out_hbm.at[base + j])

    return emb_kernel(tokens_c, pos, table)


# speedup vs baseline: 11.6169x; 1.0003x over previous
import functools

import numpy as np
import jax
import jax.numpy as jnp
from jax import lax
from jax.experimental import pallas as pl
from jax.experimental.pallas import tpu as pltpu
from jax.experimental.pallas import tpu_sc as plsc

_NC = 2
_NS = 16
_NW = _NC * _NS
_LANES = 16
_NBUF = 4


def _pos_encoding(dk: int, length: int) -> np.ndarray:
    i = np.arange(dk)
    even = (i % 2 == 0).astype(np.float64)
    power = 10000.0 ** (2.0 * i / dk)
    pos = np.arange(length)[:, None]
    ang = pos / power[None, :]
    return (np.sin(ang) * even + np.cos(ang) * (1.0 - even)).astype(np.float32)


@functools.partial(jax.jit, static_argnames=("interpret",))
def kernel(tokens, table, *, interpret=False):
    B, L = tokens.shape
    V, D = table.shape
    assert B % _NW == 0 and D % _LANES == 0
    spw = B // _NW
    assert spw % _NBUF == 0
    nch = -(-L // 128)
    assert L % nch == 0
    ch = L // nch

    pos = jnp.asarray(_pos_encoding(D, L))
    tokens_c = tokens.astype(jnp.int32).reshape(_NW, spw, nch, ch)

    mesh = plsc.VectorSubcoreMesh(
        core_axis_name="c", subcore_axis_name="s",
        num_cores=_NC, num_subcores=_NS,
    )

    @functools.partial(
        pl.kernel,
        out_type=jax.ShapeDtypeStruct((B, L, D), jnp.float32),
        mesh=mesh,
        scratch_types=[
            pltpu.VMEM((spw, nch, ch), jnp.int32),
            pltpu.VMEM((_NBUF, L, D), jnp.float32),
            pltpu.VMEM((L, D), jnp.float32),
            [pltpu.SemaphoreType.DMA for _ in range(_NBUF)],
        ],
        compiler_params=pltpu.CompilerParams(use_tc_tiling_on_sc=False),
        interpret=interpret,
    )
    def emb_kernel(tokens_hbm, pos_hbm, table_hbm, out_hbm,
                   idx_all, rows, pos_v, sem_g):
        wid = lax.axis_index("s") * _NC + lax.axis_index("c")
        base = wid * spw
        pltpu.sync_copy(pos_hbm, pos_v)
        pltpu.sync_copy(tokens_hbm.at[wid], idx_all)

        def gather(j, b, issue):
            mk = pltpu.async_copy if issue else pltpu.make_async_copy
            return [
                mk(table_hbm.at[idx_all.at[j, h]],
                   rows.at[b, pl.ds(h * ch, ch)], sem_g[b])
                for h in range(nch)
            ]

        for k in range(_NBUF - 1):
            gather(k, k, True)

        @pl.loop(0, spw, step=_NBUF)
        def _outer(jo):
            for b in range(_NBUF):
                j = jo + b
                for d in gather(j, b, False):
                    d.wait()

                @pl.loop(0, L)
                def _row(r):
                    for c in range(D // _LANES):
                        sl = pl.ds(c * _LANES, _LANES)
                        rows[b, r, sl] = rows[b, r, sl] + pos_v[r, sl]

                bp = (b - 1) % _NBUF

                @pl.when(j + _NBUF - 1 < spw)
                def _():
                    gather(j + _NBUF - 1, bp, True)

                pltpu.sync_copy(rows.at[b], out_hbm.at[base + j])

    return emb_kernel(tokens_c, pos, table)


# speedup vs baseline: 11.6483x; 1.0027x over previous
import functools

import numpy as np
import jax
import jax.numpy as jnp
from jax import lax
from jax.experimental import pallas as pl
from jax.experimental.pallas import tpu as pltpu
from jax.experimental.pallas import tpu_sc as plsc

_NC = 2
_NS = 16
_NW = _NC * _NS
_LANES = 16
_NBUF = 4


def _pos_encoding(dk: int, length: int) -> np.ndarray:
    i = np.arange(dk)
    even = (i % 2 == 0).astype(np.float64)
    power = 10000.0 ** (2.0 * i / dk)
    pos = np.arange(length)[:, None]
    ang = pos / power[None, :]
    return (np.sin(ang) * even + np.cos(ang) * (1.0 - even)).astype(np.float32)


@jax.jit
def kernel(tokens, table):
    B, L = tokens.shape
    V, D = table.shape
    assert B % _NW == 0 and D % _LANES == 0
    spw = B // _NW
    assert spw % _NBUF == 0
    nch = -(-L // 128)
    assert L % nch == 0
    ch = L // nch

    pos = jnp.asarray(_pos_encoding(D, L))
    tokens_c = tokens.astype(jnp.int32).reshape(_NW, spw, nch, ch)

    mesh = plsc.VectorSubcoreMesh(
        core_axis_name="c", subcore_axis_name="s",
        num_cores=_NC, num_subcores=_NS,
    )

    @functools.partial(
        pl.kernel,
        out_type=jax.ShapeDtypeStruct((B, L, D), jnp.float32),
        mesh=mesh,
        scratch_types=[
            pltpu.VMEM((spw, nch, ch), jnp.int32),
            pltpu.VMEM((_NBUF, L, D), jnp.float32),
            pltpu.VMEM((L, D), jnp.float32),
            [pltpu.SemaphoreType.DMA for _ in range(_NBUF)],
        ],
        compiler_params=pltpu.CompilerParams(use_tc_tiling_on_sc=False),
    )
    def emb_kernel(tokens_hbm, pos_hbm, table_hbm, out_hbm,
                   idx_all, rows, pos_v, sem_g):
        wid = lax.axis_index("s") * _NC + lax.axis_index("c")
        base = wid * spw
        pltpu.sync_copy(pos_hbm, pos_v)
        pltpu.sync_copy(tokens_hbm.at[wid], idx_all)

        def gather(j, b, issue):
            mk = pltpu.async_copy if issue else pltpu.make_async_copy
            return [
                mk(table_hbm.at[idx_all.at[j, h]],
                   rows.at[b, pl.ds(h * ch, ch)], sem_g[b])
                for h in range(nch)
            ]

        for k in range(_NBUF - 1):
            gather(k, k, True)

        @pl.loop(0, spw, step=_NBUF)
        def _outer(jo):
            for b in range(_NBUF):
                j = jo + b
                for d in gather(j, b, False):
                    d.wait()

                @pl.loop(0, L)
                def _row(r):
                    for c in range(D // _LANES):
                        sl = pl.ds(c * _LANES, _LANES)
                        rows[b, r, sl] = rows[b, r, sl] + pos_v[r, sl]

                bp = (b - 1) % _NBUF

                @pl.when(j + _NBUF - 1 < spw)
                def _():
                    gather(j + _NBUF - 1, bp, True)

                pltpu.sync_copy(rows.at[b], out_hbm.at[base + j])

    return emb_kernel(tokens_c, pos, table)
